# Initial kernel scaffold; baseline (speedup 1.0000x reference)
import jax
import jax.numpy as jnp
from jax.experimental import pallas as pl


def kernel(t_idx, pe):
    raise NotImplementedError("write your pallas kernel here")


# speedup vs baseline: 8.6652x; 8.6652x over previous
import functools

import jax
import jax.numpy as jnp
from jax import lax
from jax.experimental import pallas as pl
from jax.experimental.pallas import tpu as pltpu
from jax.experimental.pallas import tpu_sc as plsc

EMB = 128
B = 4096
T = 200
B_TOT = B * T
NC, NS = 2, 16
NW = NC * NS
PER_W = B_TOT // NW
CH = 128
NCH = PER_W // CH
NBUF = 4


def _sc_gather(idx2d, pe):
    mesh = plsc.VectorSubcoreMesh(core_axis_name="c", subcore_axis_name="s")

    @functools.partial(
        pl.kernel,
        out_type=jax.ShapeDtypeStruct((B_TOT, EMB), jnp.float32),
        mesh=mesh,
        scratch_types=[
            pltpu.VMEM((NCH, CH), jnp.int32),
            pltpu.VMEM((NBUF, CH, EMB), jnp.float32),
            pltpu.SemaphoreType.DMA((NBUF,)),
            pltpu.SemaphoreType.DMA((NBUF,)),
        ],
    )
    def k(idx_hbm, pe_hbm, out_hbm, idx_v, rows_v, gsem, ssem):
        wid = lax.axis_index("s") * NC + lax.axis_index("c")
        base = wid * PER_W

        pltpu.sync_copy(idx_hbm.at[pl.ds(wid * NCH, NCH)], idx_v)

        def gather_start(g, slot):
            pltpu.async_copy(pe_hbm.at[idx_v.at[g]], rows_v.at[slot],
                             gsem.at[slot])

        def gather_wait(g, slot):
            pltpu.make_async_copy(pe_hbm.at[idx_v.at[g]], rows_v.at[slot],
                                  gsem.at[slot]).wait()

        for b in range(NBUF):
            gather_start(b, b)

        @pl.loop(0, NCH, step=NBUF)
        def _outer(g0):
            for b in range(NBUF):
                g = g0 + b
                gather_wait(g, b)
                out_copy = pltpu.async_copy(
                    rows_v.at[b],
                    out_hbm.at[pl.ds(base + g * CH, CH)],
                    ssem.at[b])
                out_copy.wait()
                nxt = g + NBUF

                @pl.when(nxt < NCH)
                def _():
                    gather_start(nxt, b)

    return k(idx2d, pe)


def kernel(t_idx, pe):
    idx2d = t_idx.reshape(NW * NCH, CH)
    out = _sc_gather(idx2d, pe)
    return out.reshape(B, T, EMB)


# speedup vs baseline: 15.5640x; 1.7961x over previous
import functools

import jax
import jax.numpy as jnp
from jax import lax
from jax.experimental import pallas as pl
from jax.experimental.pallas import tpu as pltpu
from jax.experimental.pallas import tpu_sc as plsc

EMB = 128
B = 4096
T = 200
B_TOT = B * T
NC, NS = 2, 16
NW = NC * NS
PER_W = B_TOT // NW
CH = 128
NCH = PER_W // CH
NBUF = 4


def _sc_gather(idx2d, pe):
    mesh = plsc.VectorSubcoreMesh(core_axis_name="c", subcore_axis_name="s")

    @functools.partial(
        pl.kernel,
        out_type=jax.ShapeDtypeStruct((B_TOT, EMB), jnp.float32),
        mesh=mesh,
        scratch_types=[
            pltpu.VMEM((NCH, CH), jnp.int32),
            pltpu.VMEM((NBUF, CH, EMB), jnp.float32),
            pltpu.VMEM_SHARED((B, EMB), jnp.float32),
            pltpu.SemaphoreType.DMA((NBUF,)),
            pltpu.SemaphoreType.DMA((NBUF,)),
        ],
    )
    def k(idx_hbm, pe_hbm, out_hbm, idx_v, rows_v, pe_sh, gsem, ssem):
        sid = lax.axis_index("s")
        wid = sid * NC + lax.axis_index("c")
        base = wid * PER_W

        @pl.when(sid == 0)
        def _():
            pltpu.sync_copy(pe_hbm, pe_sh)

        pltpu.sync_copy(idx_hbm.at[pl.ds(wid * NCH, NCH)], idx_v)
        plsc.subcore_barrier()

        def gather_start(g, slot):
            pltpu.async_copy(pe_sh.at[idx_v.at[g]], rows_v.at[slot],
                             gsem.at[slot])

        def gather_wait(g, slot):
            pltpu.make_async_copy(pe_sh.at[idx_v.at[g]], rows_v.at[slot],
                                  gsem.at[slot]).wait()

        def out_start(g, slot):
            pltpu.async_copy(rows_v.at[slot],
                             out_hbm.at[pl.ds(base + g * CH, CH)],
                             ssem.at[slot])

        def out_wait(g, slot):
            pltpu.make_async_copy(rows_v.at[slot],
                                  out_hbm.at[pl.ds(base + g * CH, CH)],
                                  ssem.at[slot]).wait()

        for b in range(NBUF):
            gather_start(b, b)

        @pl.loop(0, NCH, step=NBUF)
        def _outer(g0):
            for b in range(NBUF):
                g = g0 + b
                gather_wait(g, b)
                out_start(g, b)
            for b in range(NBUF):
                g = g0 + b
                out_wait(g, b)
                nxt = g + NBUF

                @pl.when(nxt < NCH)
                def _():
                    gather_start(nxt, b)

    return k(idx2d, pe)


def kernel(t_idx, pe):
    idx2d = t_idx.reshape(NW * NCH, CH)
    out = _sc_gather(idx2d, pe)
    return out.reshape(B, T, EMB)
